# Initial kernel scaffold; baseline (speedup 1.0000x reference)
import jax
import jax.numpy as jnp
from jax.experimental import pallas as pl


def kernel(x, adj, W0, a0, W1, a1, W2, a2):
    raise NotImplementedError("write your pallas kernel here")


# speedup vs baseline: 1466.6672x; 1466.6672x over previous
import jax
import jax.numpy as jnp
from jax.experimental import pallas as pl
from jax.experimental.pallas import tpu as pltpu

N = 1024
NFEAT = 128
NHID = 64
NHEADS = 3
ALPHA = 0.2
BLK = 128
GRID = N // BLK


def _gat_kernel(x_ref, adj_ref, w_ref, a_ref, out_ref, h_ref):
    i = pl.program_id(0)

    @pl.when(i == 0)
    def _():
        xv = x_ref[...]
        for hd in range(NHEADS):
            h_ref[hd] = jnp.dot(xv, w_ref[hd], preferred_element_type=jnp.float32)

    adjb = adj_ref[...]
    rows = jax.lax.broadcasted_iota(jnp.int32, (BLK, N), 0) + i * BLK
    cols = jax.lax.broadcasted_iota(jnp.int32, (BLK, N), 1)
    mask = (adjb != 0.0) | (rows == cols)

    for hd in range(NHEADS):
        h = h_ref[hd]
        hblk = h_ref[hd, pl.ds(i * BLK, BLK), :]
        a1 = a_ref[hd, 0:NHID]
        a2 = a_ref[hd, NHID:2 * NHID]
        f1 = jnp.sum(hblk * a1[None, :], axis=1)
        f2 = jnp.sum(h * a2[None, :], axis=1)
        z = f1[:, None] + f2[None, :]
        lrelu = jnp.maximum(z, ALPHA * z)
        e = jnp.where(mask, jnp.exp(-lrelu), 0.0)
        rowsum = jnp.sum(e, axis=1)
        hp = jnp.dot(e, h, preferred_element_type=jnp.float32)
        v = hp / rowsum[:, None]
        out_ref[:, hd * NHID:(hd + 1) * NHID] = jnp.where(
            v > 0.0, v, jnp.exp(jnp.minimum(v, 0.0)) - 1.0)


def kernel(x, adj, W0, a0, W1, a1, W2, a2):
    W = jnp.stack([W0, W1, W2])
    A = jnp.stack([a0[0], a1[0], a2[0]])
    return pl.pallas_call(
        _gat_kernel,
        grid=(GRID,),
        in_specs=[
            pl.BlockSpec((N, NFEAT), lambda i: (0, 0)),
            pl.BlockSpec((BLK, N), lambda i: (i, 0)),
            pl.BlockSpec((NHEADS, NFEAT, NHID), lambda i: (0, 0, 0)),
            pl.BlockSpec((NHEADS, 2 * NHID), lambda i: (0, 0)),
        ],
        out_specs=pl.BlockSpec((BLK, NHEADS * NHID), lambda i: (i, 0)),
        out_shape=jax.ShapeDtypeStruct((N, NHEADS * NHID), jnp.float32),
        scratch_shapes=[pltpu.VMEM((NHEADS, N, NHID), jnp.float32)],
    )(x, adj, W, A)


# speedup vs baseline: 1731.1475x; 1.1803x over previous
import jax
import jax.numpy as jnp
from jax.experimental import pallas as pl
from jax.experimental.pallas import tpu as pltpu

N = 1024
NFEAT = 128
NHID = 64
NHEADS = 3
ALPHA = 0.2
BLK = 128
GRID = N // BLK


def _gat_kernel(x_ref, adj_ref, w_ref, a_ref, out_ref, hext_ref, nf1_ref, nf2_ref):
    i = pl.program_id(0)

    @pl.when(i == 0)
    def _():
        xv = x_ref[...]
        ones = jnp.ones((N, NHID), dtype=jnp.bfloat16)
        for hd in range(NHEADS):
            h = jnp.dot(xv, w_ref[hd], preferred_element_type=jnp.float32)
            hext_ref[hd, :, 0:NHID] = h.astype(jnp.bfloat16)
            hext_ref[hd, :, NHID:2 * NHID] = ones
            a1 = a_ref[hd, 0:NHID]
            a2 = a_ref[hd, NHID:2 * NHID]
            nf1_ref[hd] = -jnp.sum(h * a1[None, :], axis=1, keepdims=True)
            nf2_ref[hd] = -jnp.sum(h * a2[None, :], axis=1, keepdims=True).reshape(1, N)

    adjb = adj_ref[...]
    rows = jax.lax.broadcasted_iota(jnp.int32, (BLK, N), 0) + i * BLK
    cols = jax.lax.broadcasted_iota(jnp.int32, (BLK, N), 1)
    mask = (adjb != 0.0) | (rows == cols)

    for hd in range(NHEADS):
        nf1b = nf1_ref[hd, pl.ds(i * BLK, BLK), :]
        nf2r = nf2_ref[hd]
        t = nf1b + nf2r
        g = jnp.exp(jnp.minimum(t, ALPHA * t))
        e = jnp.where(mask, g, 0.0).astype(jnp.bfloat16)
        hp = jnp.dot(e, hext_ref[hd], preferred_element_type=jnp.float32)
        v = hp[:, 0:NHID] / hp[:, NHID:NHID + 1]
        out_ref[:, hd * NHID:(hd + 1) * NHID] = jnp.where(
            v > 0.0, v, jnp.exp(jnp.minimum(v, 0.0)) - 1.0)


def kernel(x, adj, W0, a0, W1, a1, W2, a2):
    W = jnp.stack([W0, W1, W2])
    A = jnp.stack([a0[0], a1[0], a2[0]])
    return pl.pallas_call(
        _gat_kernel,
        grid=(GRID,),
        in_specs=[
            pl.BlockSpec((N, NFEAT), lambda i: (0, 0)),
            pl.BlockSpec((BLK, N), lambda i: (i, 0)),
            pl.BlockSpec((NHEADS, NFEAT, NHID), lambda i: (0, 0, 0)),
            pl.BlockSpec((NHEADS, 2 * NHID), lambda i: (0, 0)),
        ],
        out_specs=pl.BlockSpec((BLK, NHEADS * NHID), lambda i: (i, 0)),
        out_shape=jax.ShapeDtypeStruct((N, NHEADS * NHID), jnp.float32),
        scratch_shapes=[
            pltpu.VMEM((NHEADS, N, 2 * NHID), jnp.bfloat16),
            pltpu.VMEM((NHEADS, N, 1), jnp.float32),
            pltpu.VMEM((NHEADS, 1, N), jnp.float32),
        ],
    )(x, adj, W, A)


# speedup vs baseline: 1953.5380x; 1.1285x over previous
import jax
import jax.numpy as jnp
from jax.experimental import pallas as pl
from jax.experimental.pallas import tpu as pltpu

N = 1024
NFEAT = 128
NHID = 64
NHEADS = 3
ALPHA = 0.2
BLK = 256
GRID = N // BLK


def _gat_kernel(x_ref, xt_ref, adj_ref, w_ref, wt_ref, a_ref, out_ref,
                hext_ref, nf1_ref, nf2_ref):
    i = pl.program_id(0)

    @pl.when(i == 0)
    def _():
        xv = x_ref[...]
        xt = xt_ref[...]
        ones = jnp.ones((N, NHID), dtype=jnp.bfloat16)
        for hd in range(NHEADS):
            h = jnp.dot(xv, w_ref[hd], preferred_element_type=jnp.float32)
            hext_ref[hd, :, 0:NHID] = h.astype(jnp.bfloat16)
            hext_ref[hd, :, NHID:2 * NHID] = ones
            a1 = a_ref[hd, 0:NHID]
            a2 = a_ref[hd, NHID:2 * NHID].reshape(1, NHID)
            nf1_ref[hd] = -jnp.sum(h * a1[None, :], axis=1, keepdims=True)
            c2 = jnp.dot(a2, wt_ref[hd], preferred_element_type=jnp.float32)
            nf2_ref[hd] = -jnp.dot(c2, xt, preferred_element_type=jnp.float32)

    adjb = adj_ref[...]
    rows = jax.lax.broadcasted_iota(jnp.int32, (BLK, N), 0) + i * BLK
    cols = jax.lax.broadcasted_iota(jnp.int32, (BLK, N), 1)
    mask = (adjb != 0.0) | (rows == cols)

    for hd in range(NHEADS):
        nf1b = nf1_ref[hd, pl.ds(i * BLK, BLK), :]
        nf2r = nf2_ref[hd]
        t = nf1b + nf2r
        g = jnp.exp(jnp.minimum(t, ALPHA * t))
        e = jnp.where(mask, g, 0.0).astype(jnp.bfloat16)
        hp = jnp.dot(e, hext_ref[hd], preferred_element_type=jnp.float32)
        v = hp[:, 0:NHID] / hp[:, NHID:NHID + 1]
        out_ref[:, hd * NHID:(hd + 1) * NHID] = jnp.where(
            v > 0.0, v, jnp.exp(jnp.minimum(v, 0.0)) - 1.0)


def kernel(x, adj, W0, a0, W1, a1, W2, a2):
    W = jnp.stack([W0, W1, W2])
    Wt = jnp.transpose(W, (0, 2, 1))
    A = jnp.stack([a0[0], a1[0], a2[0]])
    return pl.pallas_call(
        _gat_kernel,
        grid=(GRID,),
        in_specs=[
            pl.BlockSpec((N, NFEAT), lambda i: (0, 0)),
            pl.BlockSpec((NFEAT, N), lambda i: (0, 0)),
            pl.BlockSpec((BLK, N), lambda i: (i, 0)),
            pl.BlockSpec((NHEADS, NFEAT, NHID), lambda i: (0, 0, 0)),
            pl.BlockSpec((NHEADS, NHID, NFEAT), lambda i: (0, 0, 0)),
            pl.BlockSpec((NHEADS, 2 * NHID), lambda i: (0, 0)),
        ],
        out_specs=pl.BlockSpec((BLK, NHEADS * NHID), lambda i: (i, 0)),
        out_shape=jax.ShapeDtypeStruct((N, NHEADS * NHID), jnp.float32),
        scratch_shapes=[
            pltpu.VMEM((NHEADS, N, 2 * NHID), jnp.bfloat16),
            pltpu.VMEM((NHEADS, N, 1), jnp.float32),
            pltpu.VMEM((NHEADS, 1, N), jnp.float32),
        ],
    )(x, x.T, adj, W, Wt, A)
